# baseline (speedup 1.0000x reference)
import functools

import jax
import jax.numpy as jnp
from jax.experimental import pallas as pl
from jax.experimental.pallas import tpu as pltpu

_F32 = jnp.float32
_BF16 = jnp.bfloat16


def _shift_r(v):
    return jnp.concatenate([jnp.zeros_like(v[:, :1]), v[:, :-1]], axis=1)


def _shift_l(v):
    return jnp.concatenate([v[:, 1:], jnp.zeros_like(v[:, :1])], axis=1)


def _body(cw0_ref, cb0_ref, cw1_ref, cb1_ref,
          x_ref, eps_ref, adj_ref,
          w1_ref, b1_ref, w2_ref, b2_ref, w31_ref, b31_ref,
          w21_ref, b21_ref, w22_ref, b22_ref,
          w3_ref, b3_ref, w32_ref, b32_ref, w4_ref, b4_ref,
          g1_ref, g3_ref, g4_ref, g5_ref, fcw_ref, fcb_ref,
          mu_ref, lv_ref, rec_ref, out_ref,
          adjb_s, ta_s, tb_s, *, C, L, TM, steps, N, MC):
    def mm(a, b):
        return jnp.dot(a, b, preferred_element_type=_F32)

    def mmw(a, w_ref):
        return mm(a, w_ref[...].astype(_BF16))

    i = pl.program_id(0)

    @pl.when(i < steps)
    def _ae():
        row = i * TM
        adjb_s[pl.ds(row, TM), :] = adj_ref[...].astype(_BF16)

        xr = x_ref[...]
        pro = jnp.full((TM, L), cb0_ref[0], _F32)
        for c in range(C):
            xc = xr[:, c * L:(c + 1) * L]
            pro += (cw0_ref[c, 0] * _shift_r(xc)
                    + cw0_ref[c, 1] * xc
                    + cw0_ref[c, 2] * _shift_l(xc))
        pro = pro.astype(_BF16)
        ta_s[pl.ds(row, TM), :] = mmw(pro, g1_ref).astype(_BF16)

        h = jnp.maximum(mmw(pro, w1_ref) + b1_ref[...], 0.0).astype(_BF16)
        h = jnp.maximum(mmw(h, w2_ref) + b2_ref[...], 0.0).astype(_BF16)
        h = jnp.maximum(mmw(h, w31_ref) + b31_ref[...], 0.0).astype(_BF16)
        mu = mmw(h, w21_ref) + b21_ref[...]
        lv = mmw(h, w22_ref) + b22_ref[...]
        mu_ref[...] = mu
        lv_ref[...] = lv

        z = (eps_ref[...] * jnp.exp(0.5 * lv) + mu).astype(_BF16)
        d = jnp.maximum(mmw(z, w3_ref) + b3_ref[...], 0.0).astype(_BF16)
        d = jnp.maximum(mmw(d, w32_ref) + b32_ref[...], 0.0).astype(_BF16)
        y = mmw(d, w4_ref) + b4_ref[...]
        recon = 0.5 * (jnp.tanh(0.5 * y) + 1.0)

        r_m1 = _shift_r(recon)
        r_p1 = _shift_l(recon)
        for c in range(C):
            rec_ref[:, c, :] = (cw1_ref[c, 0] * r_m1
                                + cw1_ref[c, 1] * recon
                                + cw1_ref[c, 2] * r_p1 + cb1_ref[c])

    @pl.when(i == steps)
    def _gnn():
        stages = ((ta_s, tb_s, g3_ref, True),
                  (tb_s, ta_s, g4_ref, True),
                  (ta_s, tb_s, g5_ref, False))
        for t_in, t_out, g_ref, relu in stages:
            for m in range(0, N, MC):
                u = mm(adjb_s[pl.ds(m, MC), :], t_in[...])
                if relu:
                    u = jnp.maximum(u, 0.0)
                t_out[pl.ds(m, MC), :] = mmw(u.astype(_BF16), g_ref).astype(_BF16)
        for m in range(0, N, MC):
            u = mm(adjb_s[pl.ds(m, MC), :], tb_s[...])
            logits = mmw(u.astype(_BF16), fcw_ref) + fcb_ref[...]
            logits = logits - jnp.max(logits, axis=-1, keepdims=True)
            e = jnp.exp(logits)
            out_ref[pl.ds(m, MC), :] = e * pl.reciprocal(
                jnp.sum(e, axis=-1, keepdims=True), approx=True)


def kernel(conv0_w, conv0_b, conv1_w, conv1_b,
           fc1_w, fc1_b, fc2_w, fc2_b, fc31_w, fc31_b,
           fc21_w, fc21_b, fc22_w, fc22_b, fc3_w, fc3_b,
           fc32_w, fc32_b, fc4_w, fc4_b,
           gnn1_w, gnn3_w, gnn4_w, gnn5_w, fc_w, fc_b,
           x, adj, eps):
    N, C, L = x.shape
    CL = C * L
    n_lat = fc21_w.shape[1]
    n_clusters = fc_w.shape[1]
    Zg = gnn1_w.shape[1]
    H = fc2_w.shape[1]

    xf = x.reshape(N, CL)

    TM = 512
    steps = N // TM
    MC = 512
    vmem = pltpu.MemorySpace.VMEM
    smem = pltpu.MemorySpace.SMEM

    def full(a):
        return pl.BlockSpec(memory_space=vmem)

    def rows(block, rank3=False):
        if rank3:
            return pl.BlockSpec(block, lambda i: (jnp.minimum(i, steps - 1), 0, 0))
        return pl.BlockSpec(block, lambda i: (jnp.minimum(i, steps - 1), 0))

    weights = (fc1_w, fc1_b, fc2_w, fc2_b, fc31_w, fc31_b,
               fc21_w, fc21_b, fc22_w, fc22_b,
               fc3_w, fc3_b, fc32_w, fc32_b, fc4_w, fc4_b,
               gnn1_w, gnn3_w, gnn4_w, gnn5_w, fc_w, fc_b)
    flops = 2 * N * (12 * L + L * H + 3 * H * H + 2 * H * n_lat + n_lat * H
                     + H * L + 12 * L + L * Zg) \
        + 2 * (4 * N * N * Zg + 3 * N * Zg * Zg + N * Zg * n_clusters)
    bytes_accessed = 4 * (N * CL + N * n_lat + N * N) \
        + 4 * sum(int(a.size) for a in weights) \
        + 4 * (2 * N * n_lat + N * CL + N * n_clusters)

    mu, lv, rec, predict = pl.pallas_call(
        functools.partial(_body, C=C, L=L, TM=TM, steps=steps, N=N, MC=MC),
        grid=(steps + 1,),
        in_specs=([pl.BlockSpec(memory_space=smem)] * 4
                  + [rows((TM, CL)), rows((TM, n_lat)), rows((TM, N))]
                  + [full(a) for a in weights]),
        out_specs=(rows((TM, n_lat)), rows((TM, n_lat)),
                   rows((TM, C, L), rank3=True),
                   pl.BlockSpec((N, n_clusters), lambda i: (0, 0))),
        out_shape=(jax.ShapeDtypeStruct((N, n_lat), _F32),
                   jax.ShapeDtypeStruct((N, n_lat), _F32),
                   jax.ShapeDtypeStruct((N, C, L), _F32),
                   jax.ShapeDtypeStruct((N, n_clusters), _F32)),
        scratch_shapes=[pltpu.VMEM((N, N), _BF16),
                        pltpu.VMEM((N, Zg), _BF16),
                        pltpu.VMEM((N, Zg), _BF16)],
        compiler_params=pltpu.CompilerParams(
            dimension_semantics=("arbitrary",)),
        cost_estimate=pl.CostEstimate(flops=flops,
                                      transcendentals=N * (n_lat + L + n_clusters),
                                      bytes_accessed=bytes_accessed),
    )(conv0_w, conv0_b, conv1_w, conv1_b, xf, eps, adj, *weights)

    return rec, predict, mu, lv


# speedup vs baseline: 1.1426x; 1.1426x over previous
import functools

import jax
import jax.numpy as jnp
from jax.experimental import pallas as pl
from jax.experimental.pallas import tpu as pltpu

_F32 = jnp.float32
_BF16 = jnp.bfloat16


def _shift_r(v):
    return jnp.concatenate([jnp.zeros_like(v[:, :1]), v[:, :-1]], axis=1)


def _shift_l(v):
    return jnp.concatenate([v[:, 1:], jnp.zeros_like(v[:, :1])], axis=1)


def _body(cw0_ref, cb0_ref, cw1_ref, cb1_ref,
          x_ref, eps_ref, adj_ref,
          w1_ref, b1_ref, w2_ref, b2_ref, w31_ref, b31_ref,
          w21_ref, b21_ref, w22_ref, b22_ref,
          w3_ref, b3_ref, w32_ref, b32_ref, w4_ref, b4_ref,
          g1_ref, g3_ref, g4_ref, g5_ref, fcw_ref, fcb_ref,
          mu_ref, lv_ref, rec_ref, out_ref,
          adjb_s, ta_s, tb_s, *, C, L, TM, steps, N, MC):
    def mm(a, b):
        return jnp.dot(a, b, preferred_element_type=_F32)

    def mmw(a, w_ref):
        return mm(a, w_ref[...].astype(_BF16))

    i = pl.program_id(0)

    @pl.when(i < steps)
    def _ae():
        row = i * TM
        adjb_s[pl.ds(row, TM), :] = adj_ref[...].astype(_BF16)

        pro = jnp.full((TM, L), cb0_ref[0], _F32)
        for c in range(C):
            xc = x_ref[:, c, :]
            pro += (cw0_ref[c, 0] * _shift_r(xc)
                    + cw0_ref[c, 1] * xc
                    + cw0_ref[c, 2] * _shift_l(xc))
        pro = pro.astype(_BF16)
        ta_s[pl.ds(row, TM), :] = mmw(pro, g1_ref).astype(_BF16)

        h = jnp.maximum(mmw(pro, w1_ref) + b1_ref[...], 0.0).astype(_BF16)
        h = jnp.maximum(mmw(h, w2_ref) + b2_ref[...], 0.0).astype(_BF16)
        h = jnp.maximum(mmw(h, w31_ref) + b31_ref[...], 0.0).astype(_BF16)
        mu = mmw(h, w21_ref) + b21_ref[...]
        lv = mmw(h, w22_ref) + b22_ref[...]
        mu_ref[...] = mu
        lv_ref[...] = lv

        z = (eps_ref[...] * jnp.exp(0.5 * lv) + mu).astype(_BF16)
        d = jnp.maximum(mmw(z, w3_ref) + b3_ref[...], 0.0).astype(_BF16)
        d = jnp.maximum(mmw(d, w32_ref) + b32_ref[...], 0.0).astype(_BF16)
        y = mmw(d, w4_ref) + b4_ref[...]
        recon = 0.5 * (jnp.tanh(0.5 * y) + 1.0)

        r_m1 = _shift_r(recon)
        r_p1 = _shift_l(recon)
        for c in range(C):
            rec_ref[:, c, :] = (cw1_ref[c, 0] * r_m1
                                + cw1_ref[c, 1] * recon
                                + cw1_ref[c, 2] * r_p1 + cb1_ref[c])

    @pl.when(i == steps)
    def _gnn():
        stages = ((ta_s, tb_s, g3_ref, True),
                  (tb_s, ta_s, g4_ref, True),
                  (ta_s, tb_s, g5_ref, False))
        for t_in, t_out, g_ref, relu in stages:
            for m in range(0, N, MC):
                u = mm(adjb_s[pl.ds(m, MC), :], t_in[...])
                if relu:
                    u = jnp.maximum(u, 0.0)
                t_out[pl.ds(m, MC), :] = mmw(u.astype(_BF16), g_ref).astype(_BF16)
        for m in range(0, N, MC):
            u = mm(adjb_s[pl.ds(m, MC), :], tb_s[...])
            logits = mmw(u.astype(_BF16), fcw_ref) + fcb_ref[...]
            logits = logits - jnp.max(logits, axis=-1, keepdims=True)
            e = jnp.exp(logits)
            out_ref[pl.ds(m, MC), :] = e * pl.reciprocal(
                jnp.sum(e, axis=-1, keepdims=True), approx=True)


def kernel(conv0_w, conv0_b, conv1_w, conv1_b,
           fc1_w, fc1_b, fc2_w, fc2_b, fc31_w, fc31_b,
           fc21_w, fc21_b, fc22_w, fc22_b, fc3_w, fc3_b,
           fc32_w, fc32_b, fc4_w, fc4_b,
           gnn1_w, gnn3_w, gnn4_w, gnn5_w, fc_w, fc_b,
           x, adj, eps):
    N, C, L = x.shape
    CL = C * L
    n_lat = fc21_w.shape[1]
    n_clusters = fc_w.shape[1]
    Zg = gnn1_w.shape[1]
    H = fc2_w.shape[1]

    TM = 256
    steps = N // TM
    MC = 512
    vmem = pltpu.MemorySpace.VMEM
    smem = pltpu.MemorySpace.SMEM

    def full(a):
        return pl.BlockSpec(memory_space=vmem)

    def rows(block, rank3=False):
        if rank3:
            return pl.BlockSpec(block, lambda i: (jnp.minimum(i, steps - 1), 0, 0))
        return pl.BlockSpec(block, lambda i: (jnp.minimum(i, steps - 1), 0))

    weights = (fc1_w, fc1_b, fc2_w, fc2_b, fc31_w, fc31_b,
               fc21_w, fc21_b, fc22_w, fc22_b,
               fc3_w, fc3_b, fc32_w, fc32_b, fc4_w, fc4_b,
               gnn1_w, gnn3_w, gnn4_w, gnn5_w, fc_w, fc_b)
    flops = 2 * N * (12 * L + L * H + 3 * H * H + 2 * H * n_lat + n_lat * H
                     + H * L + 12 * L + L * Zg) \
        + 2 * (4 * N * N * Zg + 3 * N * Zg * Zg + N * Zg * n_clusters)
    bytes_accessed = 4 * (N * CL + N * n_lat + N * N) \
        + 4 * sum(int(a.size) for a in weights) \
        + 4 * (2 * N * n_lat + N * CL + N * n_clusters)

    mu, lv, rec, predict = pl.pallas_call(
        functools.partial(_body, C=C, L=L, TM=TM, steps=steps, N=N, MC=MC),
        grid=(steps + 1,),
        in_specs=([pl.BlockSpec(memory_space=smem)] * 4
                  + [rows((TM, C, L), rank3=True), rows((TM, n_lat)),
                     rows((TM, N))]
                  + [full(a) for a in weights]),
        out_specs=(rows((TM, n_lat)), rows((TM, n_lat)),
                   rows((TM, C, L), rank3=True),
                   pl.BlockSpec((N, n_clusters), lambda i: (0, 0))),
        out_shape=(jax.ShapeDtypeStruct((N, n_lat), _F32),
                   jax.ShapeDtypeStruct((N, n_lat), _F32),
                   jax.ShapeDtypeStruct((N, C, L), _F32),
                   jax.ShapeDtypeStruct((N, n_clusters), _F32)),
        scratch_shapes=[pltpu.VMEM((N, N), _BF16),
                        pltpu.VMEM((N, Zg), _BF16),
                        pltpu.VMEM((N, Zg), _BF16)],
        compiler_params=pltpu.CompilerParams(
            dimension_semantics=("arbitrary",)),
        cost_estimate=pl.CostEstimate(flops=flops,
                                      transcendentals=N * (n_lat + L + n_clusters),
                                      bytes_accessed=bytes_accessed),
    )(conv0_w, conv0_b, conv1_w, conv1_b, x, eps, adj, *weights)

    return rec, predict, mu, lv


# speedup vs baseline: 1.2236x; 1.0709x over previous
import functools

import jax
import jax.numpy as jnp
from jax.experimental import pallas as pl
from jax.experimental.pallas import tpu as pltpu

_F32 = jnp.float32
_BF16 = jnp.bfloat16


def _shift_r(v):
    return jnp.concatenate([jnp.zeros_like(v[:, :1]), v[:, :-1]], axis=1)


def _shift_l(v):
    return jnp.concatenate([v[:, 1:], jnp.zeros_like(v[:, :1])], axis=1)


def _body(cw0_ref, cb0_ref, cw1_ref, cb1_ref,
          x_ref, eps_ref, adj_ref,
          w1_ref, b1_ref, w2_ref, b2_ref, w31_ref, b31_ref,
          w21_ref, b21_ref, w22_ref, b22_ref,
          w3_ref, b3_ref, w32_ref, b32_ref, w4_ref, b4_ref,
          g1_ref, g3_ref, g4_ref, g5_ref, fcw_ref, fcb_ref,
          mu_ref, lv_ref, rec_ref, out_ref,
          adjb_s, ta_s, tb_s,
          w1c, w2c, w31c, wmlc, w3c, w32c, w4c, g1c, g3c, g4c, g5c, fcwc,
          *, C, L, TM, steps, N, MC, lat, ncl):
    def mm(a, b):
        return jnp.dot(a, b, preferred_element_type=_F32)

    i = pl.program_id(0)

    @pl.when(i == 0)
    def _cache():
        w1c[...] = w1_ref[...].astype(_BF16)
        w2c[...] = w2_ref[...].astype(_BF16)
        w31c[...] = w31_ref[...].astype(_BF16)
        wmlc[...] = jnp.zeros_like(wmlc)
        wmlc[:, pl.ds(0, lat)] = w21_ref[...].astype(_BF16)
        wmlc[:, pl.ds(128, lat)] = w22_ref[...].astype(_BF16)
        w3c[...] = w3_ref[...].astype(_BF16)
        w32c[...] = w32_ref[...].astype(_BF16)
        w4c[...] = w4_ref[...].astype(_BF16)
        g1c[...] = g1_ref[...].astype(_BF16)
        g3c[...] = g3_ref[...].astype(_BF16)
        g4c[...] = g4_ref[...].astype(_BF16)
        g5c[...] = g5_ref[...].astype(_BF16)
        fcwc[...] = fcw_ref[...].astype(_BF16)

    @pl.when(i < steps)
    def _ae():
        row = i * TM
        adjb_s[pl.ds(row, TM), :] = adj_ref[...].astype(_BF16)

        pro = jnp.full((TM, L), cb0_ref[0], _F32)
        for c in range(C):
            xc = x_ref[:, c, :]
            pro += (cw0_ref[c, 0] * _shift_r(xc)
                    + cw0_ref[c, 1] * xc
                    + cw0_ref[c, 2] * _shift_l(xc))
        pro = pro.astype(_BF16)
        ta_s[pl.ds(row, TM), :] = mm(pro, g1c[...]).astype(_BF16)

        h = jnp.maximum(mm(pro, w1c[...]) + b1_ref[...], 0.0).astype(_BF16)
        h = jnp.maximum(mm(h, w2c[...]) + b2_ref[...], 0.0).astype(_BF16)
        h = jnp.maximum(mm(h, w31c[...]) + b31_ref[...], 0.0).astype(_BF16)
        ml = mm(h, wmlc[...])
        mu = ml[:, 0:lat] + b21_ref[...]
        lv = ml[:, 128:128 + lat] + b22_ref[...]
        mu_ref[...] = mu
        lv_ref[...] = lv

        z = (eps_ref[...] * jnp.exp(0.5 * lv) + mu).astype(_BF16)
        d = jnp.maximum(mm(z, w3c[...]) + b3_ref[...], 0.0).astype(_BF16)
        d = jnp.maximum(mm(d, w32c[...]) + b32_ref[...], 0.0).astype(_BF16)
        y = mm(d, w4c[...]) + b4_ref[...]
        recon = 0.5 * (jnp.tanh(0.5 * y) + 1.0)

        r_m1 = _shift_r(recon)
        r_p1 = _shift_l(recon)
        for c in range(C):
            rec_ref[:, c, :] = (cw1_ref[c, 0] * r_m1
                                + cw1_ref[c, 1] * recon
                                + cw1_ref[c, 2] * r_p1 + cb1_ref[c])

    @pl.when(i == steps)
    def _gnn():
        for t_in, t_out, g_s in ((ta_s, tb_s, g3c),
                                 (tb_s, ta_s, g4c)):
            for m in range(0, N, MC):
                u = jnp.maximum(mm(adjb_s[pl.ds(m, MC), :], t_in[...]), 0.0)
                t_out[pl.ds(m, MC), :] = mm(u.astype(_BF16),
                                            g_s[...]).astype(_BF16)

        for m in range(0, N, MC):
            tb_s[pl.ds(m, MC), :] = mm(ta_s[pl.ds(m, MC), :],
                                       g5c[...]).astype(_BF16)
        for m in range(0, N, MC):
            ta_s[pl.ds(m, MC), :] = mm(adjb_s[pl.ds(m, MC), :],
                                       tb_s[...]).astype(_BF16)

        for m in range(0, N, MC):
            tb_s[pl.ds(m, MC), 0:ncl] = mm(ta_s[pl.ds(m, MC), :],
                                           fcwc[...]).astype(_BF16)
        for m in range(0, N, MC):
            logits = mm(adjb_s[pl.ds(m, MC), :], tb_s[:, 0:ncl]) + fcb_ref[...]
            logits = logits - jnp.max(logits, axis=-1, keepdims=True)
            e = jnp.exp(logits)
            out_ref[pl.ds(m, MC), :] = e * pl.reciprocal(
                jnp.sum(e, axis=-1, keepdims=True), approx=True)


def kernel(conv0_w, conv0_b, conv1_w, conv1_b,
           fc1_w, fc1_b, fc2_w, fc2_b, fc31_w, fc31_b,
           fc21_w, fc21_b, fc22_w, fc22_b, fc3_w, fc3_b,
           fc32_w, fc32_b, fc4_w, fc4_b,
           gnn1_w, gnn3_w, gnn4_w, gnn5_w, fc_w, fc_b,
           x, adj, eps):
    N, C, L = x.shape
    n_lat = fc21_w.shape[1]
    n_clusters = fc_w.shape[1]
    Zg = gnn1_w.shape[1]
    H = fc2_w.shape[1]

    TM = 256
    steps = N // TM
    MC = 512
    vmem = pltpu.MemorySpace.VMEM
    smem = pltpu.MemorySpace.SMEM

    def full(a):
        return pl.BlockSpec(memory_space=vmem)

    def rows(block, rank3=False):
        if rank3:
            return pl.BlockSpec(block, lambda i: (jnp.minimum(i, steps - 1), 0, 0))
        return pl.BlockSpec(block, lambda i: (jnp.minimum(i, steps - 1), 0))

    weights = (fc1_w, fc1_b, fc2_w, fc2_b, fc31_w, fc31_b,
               fc21_w, fc21_b, fc22_w, fc22_b,
               fc3_w, fc3_b, fc32_w, fc32_b, fc4_w, fc4_b,
               gnn1_w, gnn3_w, gnn4_w, gnn5_w, fc_w, fc_b)
    flops = 2 * N * (12 * L + L * H + 3 * H * H + 2 * H * n_lat + n_lat * H
                     + H * L + 12 * L + L * Zg) \
        + 2 * (4 * N * N * Zg + 3 * N * Zg * Zg + N * Zg * n_clusters)
    bytes_accessed = 4 * (N * C * L + N * n_lat + N * N) \
        + 4 * sum(int(a.size) for a in weights) \
        + 4 * (2 * N * n_lat + N * C * L + N * n_clusters)

    mu, lv, rec, predict = pl.pallas_call(
        functools.partial(_body, C=C, L=L, TM=TM, steps=steps, N=N, MC=MC,
                          lat=n_lat, ncl=n_clusters),
        grid=(steps + 1,),
        in_specs=([pl.BlockSpec(memory_space=smem)] * 4
                  + [rows((TM, C, L), rank3=True), rows((TM, n_lat)),
                     rows((TM, N))]
                  + [full(a) for a in weights]),
        out_specs=(rows((TM, n_lat)), rows((TM, n_lat)),
                   rows((TM, C, L), rank3=True),
                   pl.BlockSpec((N, n_clusters), lambda i: (0, 0))),
        out_shape=(jax.ShapeDtypeStruct((N, n_lat), _F32),
                   jax.ShapeDtypeStruct((N, n_lat), _F32),
                   jax.ShapeDtypeStruct((N, C, L), _F32),
                   jax.ShapeDtypeStruct((N, n_clusters), _F32)),
        scratch_shapes=[pltpu.VMEM((N, N), _BF16),
                        pltpu.VMEM((N, Zg), _BF16),
                        pltpu.VMEM((N, Zg), _BF16),
                        pltpu.VMEM((L, H), _BF16),
                        pltpu.VMEM((H, H), _BF16),
                        pltpu.VMEM((H, H), _BF16),
                        pltpu.VMEM((H, 256), _BF16),
                        pltpu.VMEM((n_lat, H), _BF16),
                        pltpu.VMEM((H, H), _BF16),
                        pltpu.VMEM((H, L), _BF16),
                        pltpu.VMEM((L, Zg), _BF16),
                        pltpu.VMEM((Zg, Zg), _BF16),
                        pltpu.VMEM((Zg, Zg), _BF16),
                        pltpu.VMEM((Zg, Zg), _BF16),
                        pltpu.VMEM((Zg, n_clusters), _BF16)],
        compiler_params=pltpu.CompilerParams(
            dimension_semantics=("arbitrary",)),
        cost_estimate=pl.CostEstimate(flops=flops,
                                      transcendentals=N * (n_lat + L + n_clusters),
                                      bytes_accessed=bytes_accessed),
    )(conv0_w, conv0_b, conv1_w, conv1_b, x, eps, adj, *weights)

    return rec, predict, mu, lv


# speedup vs baseline: 1.2242x; 1.0005x over previous
import functools

import jax
import jax.numpy as jnp
from jax.experimental import pallas as pl
from jax.experimental.pallas import tpu as pltpu

_F32 = jnp.float32
_BF16 = jnp.bfloat16


def _shift_r(v):
    return jnp.concatenate([jnp.zeros_like(v[:, :1]), v[:, :-1]], axis=1)


def _shift_l(v):
    return jnp.concatenate([v[:, 1:], jnp.zeros_like(v[:, :1])], axis=1)


def _body(cw0_ref, cb0_ref, cw1_ref, cb1_ref,
          x_ref, eps_ref, adj_ref,
          w1_ref, b1_ref, w2_ref, b2_ref, w31_ref, b31_ref,
          w21_ref, b21_ref, w22_ref, b22_ref,
          w3_ref, b3_ref, w32_ref, b32_ref, w4_ref, b4_ref,
          g1_ref, g3_ref, g4_ref, g5_ref, fcw_ref, fcb_ref,
          mu_ref, lv_ref, rec_ref, out_ref,
          adjb_s, ta_s, tb_s,
          w1c, w2c, w31c, wmlc, w3c, w32c, w4c, g1c, g3c, g4c, g5c, fcwc,
          *, C, L, TM, steps, N, MC, lat, ncl):
    def mm(a, b):
        return jnp.dot(a, b, preferred_element_type=_F32)

    i = pl.program_id(0)

    @pl.when(i == 0)
    def _cache():
        w1c[...] = w1_ref[...].astype(_BF16)
        w2c[...] = w2_ref[...].astype(_BF16)
        w31c[...] = w31_ref[...].astype(_BF16)
        wmlc[...] = jnp.zeros_like(wmlc)
        wmlc[:, pl.ds(0, lat)] = w21_ref[...].astype(_BF16)
        wmlc[:, pl.ds(128, lat)] = w22_ref[...].astype(_BF16)
        w3c[...] = w3_ref[...].astype(_BF16)
        w32c[...] = w32_ref[...].astype(_BF16)
        w4c[...] = w4_ref[...].astype(_BF16)
        g1c[...] = g1_ref[...].astype(_BF16)
        g3c[...] = g3_ref[...].astype(_BF16)
        g4c[...] = g4_ref[...].astype(_BF16)
        g5c[...] = g5_ref[...].astype(_BF16)
        fcwc[...] = fcw_ref[...].astype(_BF16)

    if True:
        row = i * TM
        adjb_s[pl.ds(row, TM), :] = adj_ref[...].astype(_BF16)

        pro = jnp.full((TM, L), cb0_ref[0], _F32)
        for c in range(C):
            xc = x_ref[:, c, :]
            pro += (cw0_ref[c, 0] * _shift_r(xc)
                    + cw0_ref[c, 1] * xc
                    + cw0_ref[c, 2] * _shift_l(xc))
        pro = pro.astype(_BF16)
        ta_s[pl.ds(row, TM), :] = mm(pro, g1c[...]).astype(_BF16)

        h = jnp.maximum(mm(pro, w1c[...]) + b1_ref[...], 0.0).astype(_BF16)
        h = jnp.maximum(mm(h, w2c[...]) + b2_ref[...], 0.0).astype(_BF16)
        h = jnp.maximum(mm(h, w31c[...]) + b31_ref[...], 0.0).astype(_BF16)
        ml = mm(h, wmlc[...])
        mu = ml[:, 0:lat] + b21_ref[...]
        lv = ml[:, 128:128 + lat] + b22_ref[...]
        mu_ref[...] = mu
        lv_ref[...] = lv

        z = (eps_ref[...] * jnp.exp(0.5 * lv) + mu).astype(_BF16)
        d = jnp.maximum(mm(z, w3c[...]) + b3_ref[...], 0.0).astype(_BF16)
        d = jnp.maximum(mm(d, w32c[...]) + b32_ref[...], 0.0).astype(_BF16)
        y = mm(d, w4c[...]) + b4_ref[...]
        recon = 0.5 * (jnp.tanh(0.5 * y) + 1.0)

        r_m1 = _shift_r(recon)
        r_p1 = _shift_l(recon)
        for c in range(C):
            rec_ref[:, c, :] = (cw1_ref[c, 0] * r_m1
                                + cw1_ref[c, 1] * recon
                                + cw1_ref[c, 2] * r_p1 + cb1_ref[c])

    @pl.when(i == steps - 1)
    def _gnn():
        for t_in, t_out, g_s in ((ta_s, tb_s, g3c),
                                 (tb_s, ta_s, g4c)):
            for m in range(0, N, MC):
                u = jnp.maximum(mm(adjb_s[pl.ds(m, MC), :], t_in[...]), 0.0)
                t_out[pl.ds(m, MC), :] = mm(u.astype(_BF16),
                                            g_s[...]).astype(_BF16)

        for m in range(0, N, MC):
            tb_s[pl.ds(m, MC), :] = mm(ta_s[pl.ds(m, MC), :],
                                       g5c[...]).astype(_BF16)
        for m in range(0, N, MC):
            ta_s[pl.ds(m, MC), :] = mm(adjb_s[pl.ds(m, MC), :],
                                       tb_s[...]).astype(_BF16)

        for m in range(0, N, MC):
            tb_s[pl.ds(m, MC), 0:ncl] = mm(ta_s[pl.ds(m, MC), :],
                                           fcwc[...]).astype(_BF16)
        for m in range(0, N, MC):
            logits = mm(adjb_s[pl.ds(m, MC), :], tb_s[:, 0:ncl]) + fcb_ref[...]
            logits = logits - jnp.max(logits, axis=-1, keepdims=True)
            e = jnp.exp(logits)
            out_ref[pl.ds(m, MC), :] = e * pl.reciprocal(
                jnp.sum(e, axis=-1, keepdims=True), approx=True)


def kernel(conv0_w, conv0_b, conv1_w, conv1_b,
           fc1_w, fc1_b, fc2_w, fc2_b, fc31_w, fc31_b,
           fc21_w, fc21_b, fc22_w, fc22_b, fc3_w, fc3_b,
           fc32_w, fc32_b, fc4_w, fc4_b,
           gnn1_w, gnn3_w, gnn4_w, gnn5_w, fc_w, fc_b,
           x, adj, eps):
    N, C, L = x.shape
    n_lat = fc21_w.shape[1]
    n_clusters = fc_w.shape[1]
    Zg = gnn1_w.shape[1]
    H = fc2_w.shape[1]

    TM = 256
    steps = N // TM
    MC = 512
    vmem = pltpu.MemorySpace.VMEM
    smem = pltpu.MemorySpace.SMEM

    def full(a):
        return pl.BlockSpec(memory_space=vmem)

    def rows(block, rank3=False):
        if rank3:
            return pl.BlockSpec(block, lambda i: (i, 0, 0))
        return pl.BlockSpec(block, lambda i: (i, 0))

    weights = (fc1_w, fc1_b, fc2_w, fc2_b, fc31_w, fc31_b,
               fc21_w, fc21_b, fc22_w, fc22_b,
               fc3_w, fc3_b, fc32_w, fc32_b, fc4_w, fc4_b,
               gnn1_w, gnn3_w, gnn4_w, gnn5_w, fc_w, fc_b)
    flops = 2 * N * (12 * L + L * H + 3 * H * H + 2 * H * n_lat + n_lat * H
                     + H * L + 12 * L + L * Zg) \
        + 2 * (4 * N * N * Zg + 3 * N * Zg * Zg + N * Zg * n_clusters)
    bytes_accessed = 4 * (N * C * L + N * n_lat + N * N) \
        + 4 * sum(int(a.size) for a in weights) \
        + 4 * (2 * N * n_lat + N * C * L + N * n_clusters)

    mu, lv, rec, predict = pl.pallas_call(
        functools.partial(_body, C=C, L=L, TM=TM, steps=steps, N=N, MC=MC,
                          lat=n_lat, ncl=n_clusters),
        grid=(steps,),
        in_specs=([pl.BlockSpec(memory_space=smem)] * 4
                  + [rows((TM, C, L), rank3=True), rows((TM, n_lat)),
                     rows((TM, N))]
                  + [full(a) for a in weights]),
        out_specs=(rows((TM, n_lat)), rows((TM, n_lat)),
                   rows((TM, C, L), rank3=True),
                   pl.BlockSpec((N, n_clusters), lambda i: (0, 0))),
        out_shape=(jax.ShapeDtypeStruct((N, n_lat), _F32),
                   jax.ShapeDtypeStruct((N, n_lat), _F32),
                   jax.ShapeDtypeStruct((N, C, L), _F32),
                   jax.ShapeDtypeStruct((N, n_clusters), _F32)),
        scratch_shapes=[pltpu.VMEM((N, N), _BF16),
                        pltpu.VMEM((N, Zg), _BF16),
                        pltpu.VMEM((N, Zg), _BF16),
                        pltpu.VMEM((L, H), _BF16),
                        pltpu.VMEM((H, H), _BF16),
                        pltpu.VMEM((H, H), _BF16),
                        pltpu.VMEM((H, 256), _BF16),
                        pltpu.VMEM((n_lat, H), _BF16),
                        pltpu.VMEM((H, H), _BF16),
                        pltpu.VMEM((H, L), _BF16),
                        pltpu.VMEM((L, Zg), _BF16),
                        pltpu.VMEM((Zg, Zg), _BF16),
                        pltpu.VMEM((Zg, Zg), _BF16),
                        pltpu.VMEM((Zg, Zg), _BF16),
                        pltpu.VMEM((Zg, n_clusters), _BF16)],
        compiler_params=pltpu.CompilerParams(
            dimension_semantics=("arbitrary",)),
        cost_estimate=pl.CostEstimate(flops=flops,
                                      transcendentals=N * (n_lat + L + n_clusters),
                                      bytes_accessed=bytes_accessed),
    )(conv0_w, conv0_b, conv1_w, conv1_b, x, eps, adj, *weights)

    return rec, predict, mu, lv


# speedup vs baseline: 1.2848x; 1.0495x over previous
import functools

import jax
import jax.numpy as jnp
from jax.experimental import pallas as pl
from jax.experimental.pallas import tpu as pltpu

_F32 = jnp.float32
_BF16 = jnp.bfloat16


def _shift_r(v):
    return jnp.concatenate([jnp.zeros_like(v[:, :1]), v[:, :-1]], axis=1)


def _shift_l(v):
    return jnp.concatenate([v[:, 1:], jnp.zeros_like(v[:, :1])], axis=1)


def _body(cw0_ref, cb0_ref, cw1_ref, cb1_ref,
          x_ref, eps_ref, adj_ref,
          w1_ref, b1_ref, w2_ref, b2_ref, w31_ref, b31_ref,
          w21_ref, b21_ref, w22_ref, b22_ref,
          w3_ref, b3_ref, w32_ref, b32_ref, w4_ref, b4_ref,
          g1_ref, g3_ref, g4_ref, g5_ref, fcw_ref, fcb_ref,
          mu_ref, lv_ref, rec_ref, out_ref,
          adjb_s, ta_s, tb_s,
          w1c, w2c, w31c, wmlc, w3c, w32c, w4c, g1c, g3c, g4c, g5c, fcwc,
          *, C, L, TM, steps, N, MC, lat, ncl):
    def mm(a, b):
        return jnp.dot(a, b, preferred_element_type=_F32)

    i = pl.program_id(0)

    @pl.when(i == 0)
    def _cache():
        w1c[...] = w1_ref[...].astype(_BF16)
        w2c[...] = w2_ref[...].astype(_BF16)
        w31c[...] = w31_ref[...].astype(_BF16)
        wmlc[...] = jnp.zeros_like(wmlc)
        wmlc[:, pl.ds(0, lat)] = w21_ref[...].astype(_BF16)
        wmlc[:, pl.ds(128, lat)] = w22_ref[...].astype(_BF16)
        w3c[...] = w3_ref[...].astype(_BF16)
        w32c[...] = w32_ref[...].astype(_BF16)
        w4c[...] = w4_ref[...].astype(_BF16)
        g1c[...] = g1_ref[...].astype(_BF16)
        g3c[...] = g3_ref[...].astype(_BF16)
        g4c[...] = g4_ref[...].astype(_BF16)
        g5c[...] = g5_ref[...].astype(_BF16)
        fcwc[...] = fcw_ref[...].astype(_BF16)

    if True:
        row = i * TM
        adjb_s[pl.ds(row, TM), :] = adj_ref[...].astype(_BF16)

        pro = jnp.full((TM, L), cb0_ref[0], _F32)
        for c in range(C):
            xc = x_ref[:, c, :]
            pro += (cw0_ref[c, 0] * _shift_r(xc)
                    + cw0_ref[c, 1] * xc
                    + cw0_ref[c, 2] * _shift_l(xc))
        pro = pro.astype(_BF16)
        ta_s[pl.ds(row, TM), :] = mm(pro, g1c[...]).astype(_BF16)

        h = jnp.maximum(mm(pro, w1c[...]) + b1_ref[...], 0.0).astype(_BF16)
        h = jnp.maximum(mm(h, w2c[...]) + b2_ref[...], 0.0).astype(_BF16)
        h = jnp.maximum(mm(h, w31c[...]) + b31_ref[...], 0.0).astype(_BF16)
        ml = mm(h, wmlc[...])
        mu = ml[:, 0:lat] + b21_ref[...]
        lv = ml[:, 128:128 + lat] + b22_ref[...]
        mu_ref[...] = mu
        lv_ref[...] = lv

        z = (eps_ref[...] * jnp.exp(0.5 * lv) + mu).astype(_BF16)
        d = jnp.maximum(mm(z, w3c[...]) + b3_ref[...], 0.0).astype(_BF16)
        d = jnp.maximum(mm(d, w32c[...]) + b32_ref[...], 0.0).astype(_BF16)
        y = mm(d, w4c[...]) + b4_ref[...]
        recon = 0.5 * (jnp.tanh(0.5 * y) + 1.0)

        r_m1 = _shift_r(recon)
        r_p1 = _shift_l(recon)
        for c in range(C):
            rec_ref[:, c, :] = (cw1_ref[c, 0] * r_m1
                                + cw1_ref[c, 1] * recon
                                + cw1_ref[c, 2] * r_p1 + cb1_ref[c])

    @pl.when(i == steps - 1)
    def _gnn():
        for t_in, t_out, g_s in ((ta_s, tb_s, g3c),
                                 (tb_s, ta_s, g4c)):
            for m in range(0, N, MC):
                u = jnp.maximum(mm(adjb_s[pl.ds(m, MC), :], t_in[...]), 0.0)
                t_out[pl.ds(m, MC), :] = mm(u.astype(_BF16),
                                            g_s[...]).astype(_BF16)

        for m in range(0, N, MC):
            tb_s[pl.ds(m, MC), :] = mm(ta_s[pl.ds(m, MC), :],
                                       g5c[...]).astype(_BF16)
        for m in range(0, N, MC):
            ta_s[pl.ds(m, MC), :] = mm(adjb_s[pl.ds(m, MC), :],
                                       tb_s[...]).astype(_BF16)

        for m in range(0, N, MC):
            tb_s[pl.ds(m, MC), 0:ncl] = mm(ta_s[pl.ds(m, MC), :],
                                           fcwc[...]).astype(_BF16)
        for m in range(0, N, MC):
            logits = mm(adjb_s[pl.ds(m, MC), :], tb_s[:, 0:ncl]) + fcb_ref[...]
            logits = logits - jnp.max(logits, axis=-1, keepdims=True)
            e = jnp.exp(logits)
            out_ref[pl.ds(m, MC), :] = e * pl.reciprocal(
                jnp.sum(e, axis=-1, keepdims=True), approx=True)


def kernel(conv0_w, conv0_b, conv1_w, conv1_b,
           fc1_w, fc1_b, fc2_w, fc2_b, fc31_w, fc31_b,
           fc21_w, fc21_b, fc22_w, fc22_b, fc3_w, fc3_b,
           fc32_w, fc32_b, fc4_w, fc4_b,
           gnn1_w, gnn3_w, gnn4_w, gnn5_w, fc_w, fc_b,
           x, adj, eps):
    N, C, L = x.shape
    n_lat = fc21_w.shape[1]
    n_clusters = fc_w.shape[1]
    Zg = gnn1_w.shape[1]
    H = fc2_w.shape[1]

    TM = 512
    steps = N // TM
    MC = 512
    vmem = pltpu.MemorySpace.VMEM
    smem = pltpu.MemorySpace.SMEM

    def full(a):
        return pl.BlockSpec(memory_space=vmem)

    def rows(block, rank3=False):
        if rank3:
            return pl.BlockSpec(block, lambda i: (i, 0, 0))
        return pl.BlockSpec(block, lambda i: (i, 0))

    weights = (fc1_w, fc1_b, fc2_w, fc2_b, fc31_w, fc31_b,
               fc21_w, fc21_b, fc22_w, fc22_b,
               fc3_w, fc3_b, fc32_w, fc32_b, fc4_w, fc4_b,
               gnn1_w, gnn3_w, gnn4_w, gnn5_w, fc_w, fc_b)
    flops = 2 * N * (12 * L + L * H + 3 * H * H + 2 * H * n_lat + n_lat * H
                     + H * L + 12 * L + L * Zg) \
        + 2 * (4 * N * N * Zg + 3 * N * Zg * Zg + N * Zg * n_clusters)
    bytes_accessed = 4 * (N * C * L + N * n_lat + N * N) \
        + 4 * sum(int(a.size) for a in weights) \
        + 4 * (2 * N * n_lat + N * C * L + N * n_clusters)

    mu, lv, rec, predict = pl.pallas_call(
        functools.partial(_body, C=C, L=L, TM=TM, steps=steps, N=N, MC=MC,
                          lat=n_lat, ncl=n_clusters),
        grid=(steps,),
        in_specs=([pl.BlockSpec(memory_space=smem)] * 4
                  + [rows((TM, C, L), rank3=True), rows((TM, n_lat)),
                     rows((TM, N))]
                  + [full(a) for a in weights]),
        out_specs=(rows((TM, n_lat)), rows((TM, n_lat)),
                   rows((TM, C, L), rank3=True),
                   pl.BlockSpec((N, n_clusters), lambda i: (0, 0))),
        out_shape=(jax.ShapeDtypeStruct((N, n_lat), _F32),
                   jax.ShapeDtypeStruct((N, n_lat), _F32),
                   jax.ShapeDtypeStruct((N, C, L), _F32),
                   jax.ShapeDtypeStruct((N, n_clusters), _F32)),
        scratch_shapes=[pltpu.VMEM((N, N), _BF16),
                        pltpu.VMEM((N, Zg), _BF16),
                        pltpu.VMEM((N, Zg), _BF16),
                        pltpu.VMEM((L, H), _BF16),
                        pltpu.VMEM((H, H), _BF16),
                        pltpu.VMEM((H, H), _BF16),
                        pltpu.VMEM((H, 256), _BF16),
                        pltpu.VMEM((n_lat, H), _BF16),
                        pltpu.VMEM((H, H), _BF16),
                        pltpu.VMEM((H, L), _BF16),
                        pltpu.VMEM((L, Zg), _BF16),
                        pltpu.VMEM((Zg, Zg), _BF16),
                        pltpu.VMEM((Zg, Zg), _BF16),
                        pltpu.VMEM((Zg, Zg), _BF16),
                        pltpu.VMEM((Zg, n_clusters), _BF16)],
        compiler_params=pltpu.CompilerParams(
            dimension_semantics=("arbitrary",)),
        cost_estimate=pl.CostEstimate(flops=flops,
                                      transcendentals=N * (n_lat + L + n_clusters),
                                      bytes_accessed=bytes_accessed),
    )(conv0_w, conv0_b, conv1_w, conv1_b, x, eps, adj, *weights)

    return rec, predict, mu, lv


# speedup vs baseline: 1.2924x; 1.0059x over previous
import functools

import jax
import jax.numpy as jnp
from jax.experimental import pallas as pl
from jax.experimental.pallas import tpu as pltpu

_F32 = jnp.float32
_BF16 = jnp.bfloat16


def _shift_r(v):
    return jnp.concatenate([jnp.zeros_like(v[:, :1]), v[:, :-1]], axis=1)


def _shift_l(v):
    return jnp.concatenate([v[:, 1:], jnp.zeros_like(v[:, :1])], axis=1)


def _body(cw0_ref, cb0_ref, cw1_ref, cb1_ref,
          x_ref, eps_ref, adj_ref,
          w1_ref, b1_ref, w2_ref, b2_ref, w31_ref, b31_ref,
          w21_ref, b21_ref, w22_ref, b22_ref,
          w3_ref, b3_ref, w32_ref, b32_ref, w4_ref, b4_ref,
          g1_ref, g3_ref, g4_ref, g5_ref, fcw_ref, fcb_ref,
          mu_ref, lv_ref, rec_ref, out_ref,
          adjb_s, ta_s, tb_s,
          w1c, w2c, w31c, wmlc, w3c, w32c, w4c, g1c, g3c, g4c, g5c, fcwc,
          *, C, L, TM, steps, N, MC, lat, ncl):
    def mm(a, b):
        return jnp.dot(a, b, preferred_element_type=_F32)

    i = pl.program_id(0)

    @pl.when(i == 0)
    def _cache():
        w1c[...] = w1_ref[...].astype(_BF16)
        w2c[...] = w2_ref[...].astype(_BF16)
        w31c[...] = w31_ref[...].astype(_BF16)
        wmlc[...] = jnp.zeros_like(wmlc)
        wmlc[:, pl.ds(0, lat)] = w21_ref[...].astype(_BF16)
        wmlc[:, pl.ds(128, lat)] = w22_ref[...].astype(_BF16)
        w3c[...] = w3_ref[...].astype(_BF16)
        w32c[...] = w32_ref[...].astype(_BF16)
        w4c[...] = w4_ref[...].astype(_BF16)
        g1c[...] = g1_ref[...].astype(_BF16)
        g3c[...] = g3_ref[...].astype(_BF16)
        g4c[...] = g4_ref[...].astype(_BF16)
        g5c[...] = g5_ref[...].astype(_BF16)
        fcwc[...] = fcw_ref[...].astype(_BF16)

    if True:
        row = i * TM
        adjb_s[pl.ds(row, TM), :] = adj_ref[...].astype(_BF16)

        pro = jnp.full((TM, L), cb0_ref[0], _F32)
        for c in range(C):
            xc = x_ref[:, c, :]
            pro += (cw0_ref[c, 0] * _shift_r(xc)
                    + cw0_ref[c, 1] * xc
                    + cw0_ref[c, 2] * _shift_l(xc))
        pro = pro.astype(_BF16)
        ta_s[pl.ds(row, TM), :] = mm(pro, g1c[...]).astype(_BF16)

        h = jnp.maximum(mm(pro, w1c[...]) + b1_ref[...], 0.0).astype(_BF16)
        h = jnp.maximum(mm(h, w2c[...]) + b2_ref[...], 0.0).astype(_BF16)
        h = jnp.maximum(mm(h, w31c[...]) + b31_ref[...], 0.0).astype(_BF16)
        ml = mm(h, wmlc[...])
        mu = ml[:, 0:lat] + b21_ref[...]
        lv = ml[:, 128:128 + lat] + b22_ref[...]
        mu_ref[...] = mu
        lv_ref[...] = lv

        z = (eps_ref[...] * jnp.exp(0.5 * lv) + mu).astype(_BF16)
        d = jnp.maximum(mm(z, w3c[...]) + b3_ref[...], 0.0).astype(_BF16)
        d = jnp.maximum(mm(d, w32c[...]) + b32_ref[...], 0.0).astype(_BF16)
        y = mm(d, w4c[...]) + b4_ref[...]
        recon = 0.5 * (jnp.tanh(0.5 * y) + 1.0)

        r_m1 = _shift_r(recon)
        r_p1 = _shift_l(recon)
        for c in range(C):
            rec_ref[:, c, :] = (cw1_ref[c, 0] * r_m1
                                + cw1_ref[c, 1] * recon
                                + cw1_ref[c, 2] * r_p1 + cb1_ref[c])

    @pl.when(i == steps - 1)
    def _gnn():
        for t_in, t_out, g_s in ((ta_s, tb_s, g3c),
                                 (tb_s, ta_s, g4c)):
            for m in range(0, N, MC):
                u = jnp.maximum(mm(adjb_s[pl.ds(m, MC), :], t_in[...]), 0.0)
                t_out[pl.ds(m, MC), :] = mm(u.astype(_BF16),
                                            g_s[...]).astype(_BF16)

        for m in range(0, N, MC):
            tb_s[pl.ds(m, MC), :] = mm(ta_s[pl.ds(m, MC), :],
                                       g5c[...]).astype(_BF16)
        for m in range(0, N, MC):
            ta_s[pl.ds(m, MC), :] = mm(adjb_s[pl.ds(m, MC), :],
                                       tb_s[...]).astype(_BF16)

        for m in range(0, N, MC):
            tb_s[pl.ds(m, MC), 0:ncl] = mm(ta_s[pl.ds(m, MC), :],
                                           fcwc[...]).astype(_BF16)
        for m in range(0, N, MC):
            logits = mm(adjb_s[pl.ds(m, MC), :], tb_s[:, 0:ncl]) + fcb_ref[...]
            logits = logits - jnp.max(logits, axis=-1, keepdims=True)
            e = jnp.exp(logits)
            out_ref[pl.ds(m, MC), :] = e * pl.reciprocal(
                jnp.sum(e, axis=-1, keepdims=True), approx=True)


def kernel(conv0_w, conv0_b, conv1_w, conv1_b,
           fc1_w, fc1_b, fc2_w, fc2_b, fc31_w, fc31_b,
           fc21_w, fc21_b, fc22_w, fc22_b, fc3_w, fc3_b,
           fc32_w, fc32_b, fc4_w, fc4_b,
           gnn1_w, gnn3_w, gnn4_w, gnn5_w, fc_w, fc_b,
           x, adj, eps):
    N, C, L = x.shape
    n_lat = fc21_w.shape[1]
    n_clusters = fc_w.shape[1]
    Zg = gnn1_w.shape[1]
    H = fc2_w.shape[1]

    TM = 512
    steps = N // TM
    MC = 1024
    vmem = pltpu.MemorySpace.VMEM
    smem = pltpu.MemorySpace.SMEM

    def full(a):
        return pl.BlockSpec(memory_space=vmem)

    def rows(block, rank3=False):
        if rank3:
            return pl.BlockSpec(block, lambda i: (i, 0, 0))
        return pl.BlockSpec(block, lambda i: (i, 0))

    weights = (fc1_w, fc1_b, fc2_w, fc2_b, fc31_w, fc31_b,
               fc21_w, fc21_b, fc22_w, fc22_b,
               fc3_w, fc3_b, fc32_w, fc32_b, fc4_w, fc4_b,
               gnn1_w, gnn3_w, gnn4_w, gnn5_w, fc_w, fc_b)
    flops = 2 * N * (12 * L + L * H + 3 * H * H + 2 * H * n_lat + n_lat * H
                     + H * L + 12 * L + L * Zg) \
        + 2 * (4 * N * N * Zg + 3 * N * Zg * Zg + N * Zg * n_clusters)
    bytes_accessed = 4 * (N * C * L + N * n_lat + N * N) \
        + 4 * sum(int(a.size) for a in weights) \
        + 4 * (2 * N * n_lat + N * C * L + N * n_clusters)

    mu, lv, rec, predict = pl.pallas_call(
        functools.partial(_body, C=C, L=L, TM=TM, steps=steps, N=N, MC=MC,
                          lat=n_lat, ncl=n_clusters),
        grid=(steps,),
        in_specs=([pl.BlockSpec(memory_space=smem)] * 4
                  + [rows((TM, C, L), rank3=True), rows((TM, n_lat)),
                     rows((TM, N))]
                  + [full(a) for a in weights]),
        out_specs=(rows((TM, n_lat)), rows((TM, n_lat)),
                   rows((TM, C, L), rank3=True),
                   pl.BlockSpec((N, n_clusters), lambda i: (0, 0))),
        out_shape=(jax.ShapeDtypeStruct((N, n_lat), _F32),
                   jax.ShapeDtypeStruct((N, n_lat), _F32),
                   jax.ShapeDtypeStruct((N, C, L), _F32),
                   jax.ShapeDtypeStruct((N, n_clusters), _F32)),
        scratch_shapes=[pltpu.VMEM((N, N), _BF16),
                        pltpu.VMEM((N, Zg), _BF16),
                        pltpu.VMEM((N, Zg), _BF16),
                        pltpu.VMEM((L, H), _BF16),
                        pltpu.VMEM((H, H), _BF16),
                        pltpu.VMEM((H, H), _BF16),
                        pltpu.VMEM((H, 256), _BF16),
                        pltpu.VMEM((n_lat, H), _BF16),
                        pltpu.VMEM((H, H), _BF16),
                        pltpu.VMEM((H, L), _BF16),
                        pltpu.VMEM((L, Zg), _BF16),
                        pltpu.VMEM((Zg, Zg), _BF16),
                        pltpu.VMEM((Zg, Zg), _BF16),
                        pltpu.VMEM((Zg, Zg), _BF16),
                        pltpu.VMEM((Zg, n_clusters), _BF16)],
        compiler_params=pltpu.CompilerParams(
            dimension_semantics=("arbitrary",)),
        cost_estimate=pl.CostEstimate(flops=flops,
                                      transcendentals=N * (n_lat + L + n_clusters),
                                      bytes_accessed=bytes_accessed),
    )(conv0_w, conv0_b, conv1_w, conv1_b, x, eps, adj, *weights)

    return rec, predict, mu, lv


# speedup vs baseline: 1.3460x; 1.0415x over previous
import functools

import jax
import jax.numpy as jnp
from jax.experimental import pallas as pl
from jax.experimental.pallas import tpu as pltpu

_F32 = jnp.float32
_BF16 = jnp.bfloat16


def _shift_r(v):
    return jnp.concatenate([jnp.zeros_like(v[:, :1]), v[:, :-1]], axis=1)


def _shift_l(v):
    return jnp.concatenate([v[:, 1:], jnp.zeros_like(v[:, :1])], axis=1)


def _body(cw0_ref, cb0_ref, cw1_ref, cb1_ref,
          x_ref, eps_ref, adj_ref,
          w1_ref, b1_ref, w2_ref, b2_ref, w31_ref, b31_ref,
          w21_ref, b21_ref, w22_ref, b22_ref,
          w3_ref, b3_ref, w32_ref, b32_ref, w4_ref, b4_ref,
          g1_ref, g3_ref, g4_ref, g5_ref, fcw_ref, fcb_ref,
          mu_ref, lv_ref, rec_ref, out_ref,
          adjb_s, ta_s, tb_s,
          w1c, w2c, w31c, wmlc, w3c, w32c, w4c, g1c, g3c, g4c, g5c, fcwc,
          a1c, *, C, L, TM, steps, N, MC, lat, ncl):
    def mm(a, b):
        return jnp.dot(a, b, preferred_element_type=_F32)

    i = pl.program_id(0)

    @pl.when(i == 0)
    def _cache():
        w1c[...] = w1_ref[...].astype(_BF16)
        w2c[...] = w2_ref[...].astype(_BF16)
        w31c[...] = w31_ref[...].astype(_BF16)
        wmlc[...] = jnp.zeros_like(wmlc)
        wmlc[:, pl.ds(0, lat)] = w21_ref[...].astype(_BF16)
        wmlc[:, pl.ds(128, lat)] = w22_ref[...].astype(_BF16)
        w3c[...] = w3_ref[...].astype(_BF16)
        w32c[...] = w32_ref[...].astype(_BF16)
        w4c[...] = w4_ref[...].astype(_BF16)
        g1c[...] = g1_ref[...].astype(_BF16)
        g3c[...] = g3_ref[...].astype(_BF16)
        g4c[...] = g4_ref[...].astype(_BF16)
        g5c[...] = g5_ref[...].astype(_BF16)
        fcwc[...] = fcw_ref[...].astype(_BF16)
        dj = jax.lax.broadcasted_iota(jnp.int32, (L, L), 0)
        dl = jax.lax.broadcasted_iota(jnp.int32, (L, L), 1)
        d = dj - dl
        for c in range(C):
            band = jnp.where(d == -1, cw1_ref[c, 0], 0.0)
            band = jnp.where(d == 0, cw1_ref[c, 1], band)
            band = jnp.where(d == 1, cw1_ref[c, 2], band)
            a1c[:, pl.ds(c * L, L)] = band.astype(_BF16)

    if True:
        row = i * TM
        adjb_s[pl.ds(row, TM), :] = adj_ref[...].astype(_BF16)

        pro = jnp.full((TM, L), cb0_ref[0], _F32)
        for c in range(C):
            xc = x_ref[:, c, :]
            pro += (cw0_ref[c, 0] * _shift_r(xc)
                    + cw0_ref[c, 1] * xc
                    + cw0_ref[c, 2] * _shift_l(xc))
        pro = pro.astype(_BF16)
        ta_s[pl.ds(row, TM), :] = mm(pro, g1c[...]).astype(_BF16)

        h = jnp.maximum(mm(pro, w1c[...]) + b1_ref[...], 0.0).astype(_BF16)
        h = jnp.maximum(mm(h, w2c[...]) + b2_ref[...], 0.0).astype(_BF16)
        h = jnp.maximum(mm(h, w31c[...]) + b31_ref[...], 0.0).astype(_BF16)
        ml = mm(h, wmlc[...])
        mu = ml[:, 0:lat] + b21_ref[...]
        lv = ml[:, 128:128 + lat] + b22_ref[...]
        mu_ref[...] = mu
        lv_ref[...] = lv

        z = (eps_ref[...] * jnp.exp(0.5 * lv) + mu).astype(_BF16)
        d = jnp.maximum(mm(z, w3c[...]) + b3_ref[...], 0.0).astype(_BF16)
        d = jnp.maximum(mm(d, w32c[...]) + b32_ref[...], 0.0).astype(_BF16)
        y = mm(d, w4c[...]) + b4_ref[...]
        recon = 0.5 * (jnp.tanh(0.5 * y) + 1.0)

        recon_b = recon.astype(_BF16)
        for c in range(C):
            rec_ref[:, c, :] = (mm(recon_b, a1c[:, pl.ds(c * L, L)])
                                + cb1_ref[c])

    @pl.when(i == steps - 1)
    def _gnn():
        for t_in, t_out, g_s in ((ta_s, tb_s, g3c),
                                 (tb_s, ta_s, g4c)):
            for m in range(0, N, MC):
                u = jnp.maximum(mm(adjb_s[pl.ds(m, MC), :], t_in[...]), 0.0)
                t_out[pl.ds(m, MC), :] = mm(u.astype(_BF16),
                                            g_s[...]).astype(_BF16)

        for m in range(0, N, MC):
            tb_s[pl.ds(m, MC), :] = mm(ta_s[pl.ds(m, MC), :],
                                       g5c[...]).astype(_BF16)
        for m in range(0, N, MC):
            ta_s[pl.ds(m, MC), :] = mm(adjb_s[pl.ds(m, MC), :],
                                       tb_s[...]).astype(_BF16)

        for m in range(0, N, MC):
            tb_s[pl.ds(m, MC), 0:ncl] = mm(ta_s[pl.ds(m, MC), :],
                                           fcwc[...]).astype(_BF16)
        for m in range(0, N, MC):
            logits = mm(adjb_s[pl.ds(m, MC), :], tb_s[:, 0:ncl]) + fcb_ref[...]
            logits = logits - jnp.max(logits, axis=-1, keepdims=True)
            e = jnp.exp(logits)
            out_ref[pl.ds(m, MC), :] = e * pl.reciprocal(
                jnp.sum(e, axis=-1, keepdims=True), approx=True)


def kernel(conv0_w, conv0_b, conv1_w, conv1_b,
           fc1_w, fc1_b, fc2_w, fc2_b, fc31_w, fc31_b,
           fc21_w, fc21_b, fc22_w, fc22_b, fc3_w, fc3_b,
           fc32_w, fc32_b, fc4_w, fc4_b,
           gnn1_w, gnn3_w, gnn4_w, gnn5_w, fc_w, fc_b,
           x, adj, eps):
    N, C, L = x.shape
    n_lat = fc21_w.shape[1]
    n_clusters = fc_w.shape[1]
    Zg = gnn1_w.shape[1]
    H = fc2_w.shape[1]

    TM = 512
    steps = N // TM
    MC = 1024
    vmem = pltpu.MemorySpace.VMEM
    smem = pltpu.MemorySpace.SMEM

    def full(a):
        return pl.BlockSpec(memory_space=vmem)

    def rows(block, rank3=False):
        if rank3:
            return pl.BlockSpec(block, lambda i: (i, 0, 0))
        return pl.BlockSpec(block, lambda i: (i, 0))

    weights = (fc1_w, fc1_b, fc2_w, fc2_b, fc31_w, fc31_b,
               fc21_w, fc21_b, fc22_w, fc22_b,
               fc3_w, fc3_b, fc32_w, fc32_b, fc4_w, fc4_b,
               gnn1_w, gnn3_w, gnn4_w, gnn5_w, fc_w, fc_b)
    flops = 2 * N * (12 * L + L * H + 3 * H * H + 2 * H * n_lat + n_lat * H
                     + H * L + 12 * L + L * Zg) \
        + 2 * (4 * N * N * Zg + 3 * N * Zg * Zg + N * Zg * n_clusters)
    bytes_accessed = 4 * (N * C * L + N * n_lat + N * N) \
        + 4 * sum(int(a.size) for a in weights) \
        + 4 * (2 * N * n_lat + N * C * L + N * n_clusters)

    mu, lv, rec, predict = pl.pallas_call(
        functools.partial(_body, C=C, L=L, TM=TM, steps=steps, N=N, MC=MC,
                          lat=n_lat, ncl=n_clusters),
        grid=(steps,),
        in_specs=([pl.BlockSpec(memory_space=smem)] * 4
                  + [rows((TM, C, L), rank3=True), rows((TM, n_lat)),
                     rows((TM, N))]
                  + [full(a) for a in weights]),
        out_specs=(rows((TM, n_lat)), rows((TM, n_lat)),
                   rows((TM, C, L), rank3=True),
                   pl.BlockSpec((N, n_clusters), lambda i: (0, 0))),
        out_shape=(jax.ShapeDtypeStruct((N, n_lat), _F32),
                   jax.ShapeDtypeStruct((N, n_lat), _F32),
                   jax.ShapeDtypeStruct((N, C, L), _F32),
                   jax.ShapeDtypeStruct((N, n_clusters), _F32)),
        scratch_shapes=[pltpu.VMEM((N, N), _BF16),
                        pltpu.VMEM((N, Zg), _BF16),
                        pltpu.VMEM((N, Zg), _BF16),
                        pltpu.VMEM((L, H), _BF16),
                        pltpu.VMEM((H, H), _BF16),
                        pltpu.VMEM((H, H), _BF16),
                        pltpu.VMEM((H, 256), _BF16),
                        pltpu.VMEM((n_lat, H), _BF16),
                        pltpu.VMEM((H, H), _BF16),
                        pltpu.VMEM((H, L), _BF16),
                        pltpu.VMEM((L, Zg), _BF16),
                        pltpu.VMEM((Zg, Zg), _BF16),
                        pltpu.VMEM((Zg, Zg), _BF16),
                        pltpu.VMEM((Zg, Zg), _BF16),
                        pltpu.VMEM((Zg, n_clusters), _BF16),
                        pltpu.VMEM((L, C * L), _BF16)],
        compiler_params=pltpu.CompilerParams(
            dimension_semantics=("arbitrary",)),
        cost_estimate=pl.CostEstimate(flops=flops,
                                      transcendentals=N * (n_lat + L + n_clusters),
                                      bytes_accessed=bytes_accessed),
    )(conv0_w, conv0_b, conv1_w, conv1_b, x, eps, adj, *weights)

    return rec, predict, mu, lv
